# baseline (speedup 1.0000x reference)
import functools

import jax
import jax.numpy as jnp
from jax import lax
from jax.experimental import pallas as pl
from jax.experimental.pallas import tpu as pltpu
from jax.experimental.pallas import tpu_sc as plsc

_KSET = (100.0, 100.0, 100.0)
_NS = 16
_NC = 2
_CH = 128
_HALF = 128
_DW = 16


def _tc_bw7(bcp, w7):
    def body(b_ref, w_ref, o_ref):
        o_ref[...] = jnp.dot(b_ref[...], w_ref[...],
                             preferred_element_type=jnp.float32,
                             precision=lax.Precision.HIGHEST)
    return pl.pallas_call(
        body,
        out_shape=jax.ShapeDtypeStruct((bcp.shape[0], w7.shape[1]), jnp.float32),
    )(bcp, w7)


def _tc_build_g(x, a0, a1, a2, bw, w7, scales):
    n, d = x.shape
    bn = 1024
    nb = bw.shape[0]

    def body(x_ref, a0_ref, a1_ref, a2_ref, bw_ref, w_ref, o_ref):
        xb = x_ref[...]
        g = jnp.dot(xb, w_ref[...], preferred_element_type=jnp.float32,
                    precision=lax.Precision.HIGHEST)
        col = lax.broadcasted_iota(jnp.int32, (xb.shape[0], nb), 1)
        cnt = ((a0_ref[...][:, None] == col).astype(jnp.float32) * scales[0]
               + (a1_ref[...][:, None] == col).astype(jnp.float32) * scales[1]
               + (a2_ref[...][:, None] == col).astype(jnp.float32) * scales[2])
        g = g + jnp.dot(cnt, bw_ref[...], preferred_element_type=jnp.float32,
                        precision=lax.Precision.HIGHEST)
        o_ref[0] = g[:, :_HALF]
        o_ref[1] = g[:, _HALF:]

    return pl.pallas_call(
        body,
        grid=(pl.cdiv(n, bn),),
        in_specs=[
            pl.BlockSpec((bn, d), lambda i: (i, 0)),
            pl.BlockSpec((bn,), lambda i: (i,)),
            pl.BlockSpec((bn,), lambda i: (i,)),
            pl.BlockSpec((bn,), lambda i: (i,)),
            pl.BlockSpec((nb, d), lambda i: (0, 0)),
            pl.BlockSpec((d, d), lambda i: (0, 0)),
        ],
        out_specs=pl.BlockSpec((2, bn, _HALF), lambda i: (0, i, 0)),
        out_shape=jax.ShapeDtypeStruct((2, n, _HALF), jnp.float32),
    )(x, a0, a1, a2, bw, w7)


def _sc_aggregate(g2, srcs4, dst3, zacc, zdeg1, ones1, n):
    zr = 8 * (-(-(n + 1) // (8 * _NS)))
    nacc = _NS * zr
    dzr = 128 * (-(-zr // 128))
    ndeg = _NS * dzr
    nch = srcs4.shape[2]
    mesh = plsc.VectorSubcoreMesh(core_axis_name="c", subcore_axis_name="s")

    @functools.partial(
        pl.kernel,
        out_type=(jax.ShapeDtypeStruct((_NC, nacc, _HALF), jnp.float32),
                  jax.ShapeDtypeStruct((_NC * ndeg,), jnp.float32)),
        mesh=mesh,
        scratch_types=[
            pltpu.VMEM((nch, _CH), jnp.int32),
            pltpu.VMEM((nch, _CH), jnp.int32),
            pltpu.VMEM((_CH, _HALF), jnp.float32),
            pltpu.VMEM((_CH,), jnp.float32),
            pltpu.VMEM_SHARED((nacc, _HALF), jnp.float32),
            pltpu.VMEM_SHARED((ndeg,), jnp.float32),
            pltpu.SemaphoreType.DMA,
        ],
    )
    def k(g2_hbm, srcs4_hbm, dst3_hbm, zacc_hbm, zdeg1_hbm, ones1_hbm,
          acc_out, deg_out, sidx_all, didx_all, rows, ones_v,
          acc_sh, deg_sh, sem):
        c = lax.axis_index("c")
        s = lax.axis_index("s")
        pltpu.sync_copy(zacc_hbm, acc_sh.at[pl.ds(s * zr, zr)])
        pltpu.sync_copy(zdeg1_hbm, deg_sh.at[pl.ds(s * dzr, dzr)])
        pltpu.sync_copy(ones1_hbm, ones_v)
        pltpu.sync_copy(srcs4_hbm.at[c, s], sidx_all)
        pltpu.sync_copy(dst3_hbm.at[s], didx_all)
        plsc.subcore_barrier()

        def body(b, carry):
            pltpu.async_copy(g2_hbm.at[sidx_all.at[b]], rows, sem).wait()
            pltpu.sync_copy(rows, acc_sh.at[didx_all.at[b]], add=True)

            @pl.when(lax.rem(b, _NC) == c)
            def _deg():
                pltpu.sync_copy(ones_v, deg_sh.at[didx_all.at[b]], add=True)
            return carry
        lax.fori_loop(0, nch, body, 0)
        plsc.subcore_barrier()

        pltpu.sync_copy(acc_sh.at[pl.ds(s * zr, zr)],
                        acc_out.at[c, pl.ds(s * zr, zr)])
        pltpu.sync_copy(deg_sh.at[pl.ds(s * dzr, dzr)],
                        deg_out.at[pl.ds(c * ndeg + s * dzr, dzr)])

    return k(g2, srcs4, dst3, zacc, zdeg1, ones1)


def _tc_divide(acc, deg, n):
    bn = 1024

    def body(a_ref, d_ref, o_ref):
        r = (1.0 / jnp.maximum(d_ref[0] + d_ref[1], 1.0))[:, None]
        o_ref[:, 0:_HALF] = a_ref[0] * r
        o_ref[:, _HALF:] = a_ref[1] * r

    return pl.pallas_call(
        body,
        grid=(pl.cdiv(n, bn),),
        in_specs=[
            pl.BlockSpec((2, bn, _HALF), lambda i: (0, i, 0)),
            pl.BlockSpec((2, bn), lambda i: (0, i)),
        ],
        out_specs=pl.BlockSpec((bn, 2 * _HALF), lambda i: (i, 0)),
        out_shape=jax.ShapeDtypeStruct((n, 2 * _HALF), jnp.float32),
    )(acc, deg)


def kernel(x, edge_index, bc_feature, bc_assigment, W7):
    n, d = x.shape
    e = edge_index.shape[1]
    nblk = bc_feature.shape[0]

    bcp = jnp.zeros((128, d), jnp.float32).at[:nblk].set(bc_feature)
    bw = _tc_bw7(bcp, W7)
    a = bc_assigment.reshape(len(_KSET), n).astype(jnp.int32)
    scales = tuple(kk / n for kk in _KSET)
    g = _tc_build_g(x, a[0], a[1], a[2], bw, W7, scales)
    g2 = g.reshape(2 * n, _HALF)

    src = edge_index[0].astype(jnp.int32)
    dst = edge_index[1].astype(jnp.int32)
    nb_t = 8 * (-(-(-(-e // (_NS * _CH))) // 8))
    ept = nb_t * _CH
    pad = ept * _NS - e
    src_p = jnp.concatenate([src, jnp.zeros((pad,), jnp.int32)])
    dst_p = jnp.concatenate([dst, jnp.full((pad,), n, jnp.int32)])
    src3 = src_p.reshape(_NS, ept // _CH, _CH)
    srcs4 = jnp.stack([src3, src3 + n])
    dst3 = dst_p.reshape(_NS, ept // _CH, _CH)

    zr = 8 * (-(-(n + 1) // (8 * _NS)))
    zacc = jnp.zeros((zr, _HALF), jnp.float32)
    zdeg1 = jnp.zeros((128 * (-(-zr // 128)),), jnp.float32)
    ones1 = jnp.ones((_CH,), jnp.float32)

    acc, deg = _sc_aggregate(g2, srcs4, dst3, zacc, zdeg1, ones1, n)
    deg = deg.reshape(_NC, deg.shape[0] // _NC)

    return _tc_divide(acc, deg, n)


# speedup vs baseline: 1.1858x; 1.1858x over previous
import functools

import jax
import jax.numpy as jnp
from jax import lax
from jax.experimental import pallas as pl
from jax.experimental.pallas import tpu as pltpu
from jax.experimental.pallas import tpu_sc as plsc

_KSET = (100.0, 100.0, 100.0)
_NS = 16
_NC = 2
_CH = 128
_HALF = 128
_DW = 16


def _tc_bw7(bcp, w7):
    def body(b_ref, w_ref, o_ref):
        o_ref[...] = jnp.dot(b_ref[...], w_ref[...],
                             preferred_element_type=jnp.float32,
                             precision=lax.Precision.HIGHEST)
    return pl.pallas_call(
        body,
        out_shape=jax.ShapeDtypeStruct((bcp.shape[0], w7.shape[1]), jnp.float32),
    )(bcp, w7)


def _tc_build_g(x, a0, a1, a2, bw, w7, scales):
    n, d = x.shape
    bn = 1024
    nb = bw.shape[0]

    def body(x_ref, a0_ref, a1_ref, a2_ref, bw_ref, w_ref, o_ref):
        xb = x_ref[...]
        g = jnp.dot(xb, w_ref[...], preferred_element_type=jnp.float32,
                    precision=lax.Precision.HIGHEST)
        col = lax.broadcasted_iota(jnp.int32, (xb.shape[0], nb), 1)
        cnt = ((a0_ref[...][:, None] == col).astype(jnp.float32) * scales[0]
               + (a1_ref[...][:, None] == col).astype(jnp.float32) * scales[1]
               + (a2_ref[...][:, None] == col).astype(jnp.float32) * scales[2])
        g = g + jnp.dot(cnt, bw_ref[...], preferred_element_type=jnp.float32,
                        precision=lax.Precision.HIGHEST)
        o_ref[0] = g[:, :_HALF]
        o_ref[1] = g[:, _HALF:]

    return pl.pallas_call(
        body,
        grid=(pl.cdiv(n, bn),),
        in_specs=[
            pl.BlockSpec((bn, d), lambda i: (i, 0)),
            pl.BlockSpec((bn,), lambda i: (i,)),
            pl.BlockSpec((bn,), lambda i: (i,)),
            pl.BlockSpec((bn,), lambda i: (i,)),
            pl.BlockSpec((nb, d), lambda i: (0, 0)),
            pl.BlockSpec((d, d), lambda i: (0, 0)),
        ],
        out_specs=pl.BlockSpec((2, bn, _HALF), lambda i: (0, i, 0)),
        out_shape=jax.ShapeDtypeStruct((2, n, _HALF), jnp.float32),
    )(x, a0, a1, a2, bw, w7)


def _sc_aggregate(g2, srcs4, dst3, zacc, zdeg1, ones1, n):
    zr = 8 * (-(-(n + 1) // (8 * _NS)))
    nacc = _NS * zr
    dzr = 128 * (-(-zr // 128))
    ndeg = _NS * dzr
    nch = srcs4.shape[2]
    mesh = plsc.VectorSubcoreMesh(core_axis_name="c", subcore_axis_name="s")

    @functools.partial(
        pl.kernel,
        out_type=(jax.ShapeDtypeStruct((_NC, nacc, _HALF), jnp.float32),
                  jax.ShapeDtypeStruct((_NC * ndeg,), jnp.float32)),
        mesh=mesh,
        scratch_types=[
            pltpu.VMEM((_CH,), jnp.int32),
            pltpu.VMEM((_CH,), jnp.int32),
            pltpu.VMEM((_CH,), jnp.int32),
            pltpu.VMEM((_CH,), jnp.int32),
            pltpu.VMEM((_CH, _HALF), jnp.float32),
            pltpu.VMEM((_CH, _HALF), jnp.float32),
            pltpu.VMEM((_CH,), jnp.float32),
            pltpu.VMEM_SHARED((nacc, _HALF), jnp.float32),
            pltpu.VMEM_SHARED((ndeg,), jnp.float32),
            pltpu.SemaphoreType.DMA,
            pltpu.SemaphoreType.DMA,
        ],
    )
    def k(g2_hbm, srcs4_hbm, dst3_hbm, zacc_hbm, zdeg1_hbm, ones1_hbm,
          acc_out, deg_out, sidx0, sidx1, didx0, didx1, rows0, rows1,
          ones_v, acc_sh, deg_sh, sem0, sem1):
        c = lax.axis_index("c")
        s = lax.axis_index("s")
        pltpu.sync_copy(zacc_hbm, acc_sh.at[pl.ds(s * zr, zr)])
        pltpu.sync_copy(zdeg1_hbm, deg_sh.at[pl.ds(s * dzr, dzr)])
        pltpu.sync_copy(ones1_hbm, ones_v)
        plsc.subcore_barrier()

        pltpu.sync_copy(srcs4_hbm.at[c, s, 0], sidx0)
        pltpu.sync_copy(dst3_hbm.at[s, 0], didx0)
        pltpu.async_copy(g2_hbm.at[sidx0], rows0, sem0)

        def body(i, carry):
            b0 = 2 * i
            b1 = b0 + 1
            pltpu.sync_copy(srcs4_hbm.at[c, s, b1], sidx1)
            pltpu.sync_copy(dst3_hbm.at[s, b1], didx1)
            pltpu.async_copy(g2_hbm.at[sidx1], rows1, sem1)
            pltpu.make_async_copy(g2_hbm.at[pl.ds(0, _CH)], rows0, sem0).wait()
            pltpu.sync_copy(rows0, acc_sh.at[didx0], add=True)

            @pl.when(c == 0)
            def _deg0():
                pltpu.sync_copy(ones_v, deg_sh.at[didx0], add=True)

            @pl.when(b0 + 2 < nch)
            def _next():
                pltpu.sync_copy(srcs4_hbm.at[c, s, b0 + 2], sidx0)
                pltpu.sync_copy(dst3_hbm.at[s, b0 + 2], didx0)
                pltpu.async_copy(g2_hbm.at[sidx0], rows0, sem0)

            pltpu.make_async_copy(g2_hbm.at[pl.ds(0, _CH)], rows1, sem1).wait()
            pltpu.sync_copy(rows1, acc_sh.at[didx1], add=True)

            @pl.when(c == 1)
            def _deg1():
                pltpu.sync_copy(ones_v, deg_sh.at[didx1], add=True)
            return carry
        lax.fori_loop(0, nch // 2, body, 0)
        plsc.subcore_barrier()

        pltpu.sync_copy(acc_sh.at[pl.ds(s * zr, zr)],
                        acc_out.at[c, pl.ds(s * zr, zr)])
        pltpu.sync_copy(deg_sh.at[pl.ds(s * dzr, dzr)],
                        deg_out.at[pl.ds(c * ndeg + s * dzr, dzr)])

    return k(g2, srcs4, dst3, zacc, zdeg1, ones1)


def _tc_divide(acc, deg, n):
    bn = 1024

    def body(a_ref, d_ref, o_ref):
        r = (1.0 / jnp.maximum(d_ref[0] + d_ref[1], 1.0))[:, None]
        o_ref[:, 0:_HALF] = a_ref[0] * r
        o_ref[:, _HALF:] = a_ref[1] * r

    return pl.pallas_call(
        body,
        grid=(pl.cdiv(n, bn),),
        in_specs=[
            pl.BlockSpec((2, bn, _HALF), lambda i: (0, i, 0)),
            pl.BlockSpec((2, bn), lambda i: (0, i)),
        ],
        out_specs=pl.BlockSpec((bn, 2 * _HALF), lambda i: (i, 0)),
        out_shape=jax.ShapeDtypeStruct((n, 2 * _HALF), jnp.float32),
    )(acc, deg)


def kernel(x, edge_index, bc_feature, bc_assigment, W7):
    n, d = x.shape
    e = edge_index.shape[1]
    nblk = bc_feature.shape[0]

    bcp = jnp.zeros((128, d), jnp.float32).at[:nblk].set(bc_feature)
    bw = _tc_bw7(bcp, W7)
    a = bc_assigment.reshape(len(_KSET), n).astype(jnp.int32)
    scales = tuple(kk / n for kk in _KSET)
    g = _tc_build_g(x, a[0], a[1], a[2], bw, W7, scales)
    g2 = g.reshape(2 * n, _HALF)

    src = edge_index[0].astype(jnp.int32)
    dst = edge_index[1].astype(jnp.int32)
    nb_t = 8 * (-(-(-(-e // (_NS * _CH))) // 8))
    ept = nb_t * _CH
    pad = ept * _NS - e
    src_p = jnp.concatenate([src, jnp.zeros((pad,), jnp.int32)])
    dst_p = jnp.concatenate([dst, jnp.full((pad,), n, jnp.int32)])
    src3 = src_p.reshape(_NS, ept // _CH, _CH)
    srcs4 = jnp.stack([src3, src3 + n])
    dst3 = dst_p.reshape(_NS, ept // _CH, _CH)

    zr = 8 * (-(-(n + 1) // (8 * _NS)))
    zacc = jnp.zeros((zr, _HALF), jnp.float32)
    zdeg1 = jnp.zeros((128 * (-(-zr // 128)),), jnp.float32)
    ones1 = jnp.ones((_CH,), jnp.float32)

    acc, deg = _sc_aggregate(g2, srcs4, dst3, zacc, zdeg1, ones1, n)
    deg = deg.reshape(_NC, deg.shape[0] // _NC)

    return _tc_divide(acc, deg, n)
